# Initial kernel scaffold; baseline (speedup 1.0000x reference)
import jax
import jax.numpy as jnp
from jax.experimental import pallas as pl


def kernel(inputs, table):
    raise NotImplementedError("write your pallas kernel here")


# speedup vs baseline: 3.6167x; 3.6167x over previous
import functools

import jax
import jax.numpy as jnp
from jax import lax
from jax.experimental import pallas as pl
from jax.experimental.pallas import tpu as pltpu
from jax.experimental.pallas import tpu_sc as plsc

_EMB = 128
_CHUNK = 128
_NBUF = 2


@functools.lru_cache(maxsize=None)
def _build(total: int, vocab: int):
    info = plsc.get_sparse_core_info()
    nc, ns = info.num_cores, info.num_subcores
    nw = nc * ns
    assert total % (nw * _CHUNK) == 0
    n_chunks = total // (nw * _CHUNK)
    assert n_chunks % _NBUF == 0

    mesh = plsc.VectorSubcoreMesh(core_axis_name="c", subcore_axis_name="s")

    @functools.partial(
        pl.kernel,
        out_type=jax.ShapeDtypeStruct((total, _EMB), jnp.float32),
        mesh=mesh,
        scratch_types=[
            pltpu.VMEM((n_chunks * _CHUNK,), jnp.int32),
            pltpu.VMEM((_NBUF, _CHUNK, _EMB), jnp.float32),
            pltpu.SemaphoreType.DMA,
            pltpu.SemaphoreType.DMA,
        ],
    )
    def gather_kernel(idx_hbm, table_hbm, out_hbm, idx_v, rows_v, gsem, osem):
        wid = lax.axis_index("s") * nc + lax.axis_index("c")
        row0 = wid * n_chunks

        pltpu.sync_copy(idx_hbm.at[pl.ds(row0 * _CHUNK, n_chunks * _CHUNK)], idx_v)

        pltpu.async_copy(
            table_hbm.at[idx_v.at[pl.ds(0, _CHUNK)]], rows_v.at[0], gsem
        )

        def pair_body(i, _):
            for b in range(_NBUF):
                j = _NBUF * i + b
                nxt = (b + 1) % _NBUF

                @pl.when(j > 0)
                def _():
                    pltpu.make_async_copy(
                        rows_v.at[nxt],
                        out_hbm.at[pl.ds((row0 + j - 1) * _CHUNK, _CHUNK)],
                        osem,
                    ).wait()

                @pl.when(j + 1 < n_chunks)
                def _():
                    pltpu.async_copy(
                        table_hbm.at[idx_v.at[pl.ds((j + 1) * _CHUNK, _CHUNK)]],
                        rows_v.at[nxt],
                        gsem,
                    )

                pltpu.make_async_copy(
                    table_hbm.at[idx_v.at[pl.ds(j * _CHUNK, _CHUNK)]],
                    rows_v.at[b],
                    gsem,
                ).wait()
                pltpu.async_copy(
                    rows_v.at[b],
                    out_hbm.at[pl.ds((row0 + j) * _CHUNK, _CHUNK)],
                    osem,
                )
            return ()

        lax.fori_loop(0, n_chunks // _NBUF, pair_body, ())

        pltpu.make_async_copy(
            rows_v.at[(n_chunks - 1) % _NBUF],
            out_hbm.at[pl.ds((row0 + n_chunks - 1) * _CHUNK, _CHUNK)],
            osem,
        ).wait()

    return gather_kernel


def kernel(inputs, table):
    b, s, n = inputs.shape
    total = b * s * n
    idx1d = inputs.reshape(total).astype(jnp.int32)
    out = _build(total, table.shape[0])(idx1d, table)
    return out.reshape(b, s, n, table.shape[1])
